# Initial kernel scaffold; baseline (speedup 1.0000x reference)
import jax
import jax.numpy as jnp
from jax.experimental import pallas as pl


def kernel(pred_targets, gt_boxes, gt_labels):
    raise NotImplementedError("write your pallas kernel here")


# speedup vs baseline: 3.6730x; 3.6730x over previous
import functools

import jax
import jax.numpy as jnp
import numpy as np
from jax import lax
from jax.experimental import pallas as pl

_ANCHORS = np.array(
    [[1.3221, 1.73145], [3.19275, 4.00944], [5.05587, 8.09892],
     [9.47112, 4.84053], [11.2364, 10.0071]], dtype=np.float32)
_A = 5
_C = 20
_S = 13
_LOC = _S * _S * _A


def _anchor_rows_np():
    ys, xs = np.meshgrid(np.arange(_S, dtype=np.float32),
                         np.arange(_S, dtype=np.float32), indexing='ij')
    xy = np.stack([xs, ys], axis=-1) + 0.5
    xy = np.broadcast_to(xy[:, :, None, :], (_S, _S, _A, 2))
    wh = np.broadcast_to(_ANCHORS[None, None, :, :], (_S, _S, _A, 2))
    cat = np.concatenate([xy, wh], axis=-1).reshape(_LOC, 4)
    return np.ascontiguousarray(cat.T)


_ANC_ROWS = _anchor_rows_np()


def _tc_kernel(pred_ref, gt_ref, lab_ref, anc_ref, out_ref):
    b = pl.program_id(0)

    @pl.when(b == 0)
    def _init():
        out_ref[...] = jnp.zeros_like(out_ref)

    pt = pred_ref[0]
    gt = gt_ref[0]
    labf = lab_ref[0]
    anc = anc_ref[...]

    pxy = jax.nn.sigmoid(pt[0:2])
    pwh = jnp.exp(pt[2:4])
    pconf = jax.nn.sigmoid(pt[4:5])

    cpx = anc[0:1] + pxy[0:1]
    cpy = anc[1:2] + pxy[1:2]
    cpw = anc[2:3] * pwh[0:1]
    cph = anc[3:4] * pwh[1:2]
    px1 = cpx - cpw * 0.5
    py1 = cpy - cph * 0.5
    px2 = cpx + cpw * 0.5
    py2 = cpy + cph * 0.5
    area_p = (px2 - px1) * (py2 - py1)

    gx1 = gt[:, 0:1] * float(_S)
    gy1 = gt[:, 1:2] * float(_S)
    gx2 = gt[:, 2:3] * float(_S)
    gy2 = gt[:, 3:4] * float(_S)
    area_g = (gx2 - gx1) * (gy2 - gy1)

    tlx = jnp.maximum(px1, gx1)
    tly = jnp.maximum(py1, gy1)
    brx = jnp.minimum(px2, gx2)
    bry = jnp.minimum(py2, gy2)
    wx = jnp.maximum(brx - tlx, 0.0)
    wy = jnp.maximum(bry - tly, 0.0)
    inter = wx * wy
    iou = inter / (area_p + area_g - inter)
    gt_conf = jnp.max(iou, axis=0, keepdims=True)

    d = gt_conf - pconf
    dense_sum = jnp.sum(d * d)

    bx = (gx1 + gx2) * 0.5
    by = (gy1 + gy2) * 0.5
    bw = gx2 - gx1
    bh = gy2 - gy1
    cxf = jnp.floor(bx)
    cyf = jnp.floor(by)
    tx = bx - cxf
    ty = by - cyf

    best = jnp.full_like(bx, -1.0)
    jbest = jnp.zeros_like(bx)
    aw_j = jnp.zeros_like(bx)
    ah_j = jnp.zeros_like(bx)
    for a in range(_A):
        aw = float(_ANCHORS[a, 0])
        ah = float(_ANCHORS[a, 1])
        ax1 = cxf + 0.5 - aw * 0.5
        ay1 = cyf + 0.5 - ah * 0.5
        ax2 = cxf + 0.5 + aw * 0.5
        ay2 = cyf + 0.5 + ah * 0.5
        itlx = jnp.maximum(ax1, gx1)
        itly = jnp.maximum(ay1, gy1)
        ibrx = jnp.minimum(ax2, gx2)
        ibry = jnp.minimum(ay2, gy2)
        iwx = jnp.maximum(ibrx - itlx, 0.0)
        iwy = jnp.maximum(ibry - itly, 0.0)
        ai = iwx * iwy
        aiou = ai / (aw * ah + area_g - ai)
        take = aiou > best
        best = jnp.maximum(best, aiou)
        jbest = jnp.where(take, float(a), jbest)
        aw_j = jnp.where(take, aw, aw_j)
        ah_j = jnp.where(take, ah, ah_j)

    loc = (cyf * float(_S) + cxf) * float(_A) + jbest
    gw = bw / aw_j
    gh = bh / ah_j
    clsf = labf - 1.0

    loc_row = lax.dot_general(loc, jnp.eye(8, dtype=jnp.float32),
                              (((0,), (0,)), ((), ())))
    cls_row = lax.dot_general(clsf, jnp.eye(8, dtype=jnp.float32),
                              (((0,), (0,)), ((), ())))
    eq = (loc == loc_row)
    eq_pair = eq & (clsf == cls_row)
    n_iota = lax.broadcasted_iota(jnp.int32, (8, 8), 0).astype(jnp.float32)
    m_iota = lax.broadcasted_iota(jnp.int32, (8, 8), 1).astype(jnp.float32)
    later = m_iota > n_iota
    any_later_eq = jnp.sum(jnp.where(eq & later, 1.0, 0.0), axis=1,
                           keepdims=True)
    any_later_pair = jnp.sum(jnp.where(eq_pair & later, 1.0, 0.0), axis=1,
                             keepdims=True)
    winner = jnp.where(any_later_eq > 0.0, 0.0, 1.0)
    uniq = jnp.where(any_later_pair > 0.0, 0.0, 1.0)

    lane = lax.broadcasted_iota(jnp.int32, (8, _LOC), 1).astype(jnp.float32)
    onehot = jnp.where(lane == loc, 1.0, 0.0)
    g = lax.dot_general(onehot, pt, (((1,), (1,)), ((), ())))
    gtc = lax.dot_general(onehot, gt_conf, (((1,), (1,)), ((), ())))

    gxy = jax.nn.sigmoid(g[:, 0:2])
    gwh = jnp.exp(g[:, 2:4])
    gconf = jax.nn.sigmoid(g[:, 4:5])
    gcls = g[:, 5:25]

    txty = jnp.concatenate([tx, ty], axis=1)
    xy_s = jnp.sum(winner * (txty - gxy) ** 2)

    gtwh = jnp.concatenate([gw, gh], axis=1)
    wh_s = jnp.sum(winner * (jnp.sqrt(gtwh) - jnp.sqrt(gwh)) ** 2)

    conf_s = jnp.sum(winner * (gtc - gconf) ** 2)

    cmax = jnp.max(gcls, axis=1, keepdims=True)
    lse = jnp.log(jnp.sum(jnp.exp(gcls - cmax), axis=1, keepdims=True)) + cmax
    c_iota = lax.broadcasted_iota(jnp.int32, (8, _C), 1).astype(jnp.float32)
    sel = jnp.sum(jnp.where(c_iota == clsf, gcls, 0.0), axis=1, keepdims=True)
    cls_s = jnp.sum(uniq * (lse - sel))

    s_iota = lax.broadcasted_iota(jnp.int32, (8, 128), 0)
    l_iota = lax.broadcasted_iota(jnp.int32, (8, 128), 1)
    vals = (jnp.where(s_iota == 0, xy_s, 0.0)
            + jnp.where(s_iota == 1, wh_s, 0.0)
            + jnp.where(s_iota == 2, conf_s, 0.0)
            + jnp.where(s_iota == 3, dense_sum, 0.0)
            + jnp.where(s_iota == 4, cls_s, 0.0))
    out_ref[...] += jnp.where(l_iota == 0, vals, 0.0)


@jax.jit
def _run(pred_targets, gt_boxes, labf):
    B = pred_targets.shape[0]
    predT = pred_targets.reshape(B, _S, _S, _A, 5 + _C)
    predT = predT.transpose(0, 4, 1, 2, 3).reshape(B, 5 + _C, _LOC)
    anc = jnp.asarray(_ANC_ROWS)

    out = pl.pallas_call(
        _tc_kernel,
        grid=(B,),
        in_specs=[
            pl.BlockSpec((1, 5 + _C, _LOC), lambda b: (b, 0, 0)),
            pl.BlockSpec((1, 8, 4), lambda b: (b, 0, 0)),
            pl.BlockSpec((1, 8, 1), lambda b: (b, 0, 0)),
            pl.BlockSpec((4, _LOC), lambda b: (0, 0)),
        ],
        out_specs=pl.BlockSpec((8, 128), lambda b: (0, 0)),
        out_shape=jax.ShapeDtypeStruct((8, 128), jnp.float32),
    )(predT, gt_boxes, labf, anc)

    o = out[:, 0]
    l1 = 5.0 * o[0]
    l2 = 5.0 * o[1]
    l3 = o[2]
    l4 = 0.5 * (o[3] - o[2])
    l5 = o[4]
    total = l1 + l2 + l3 + l4 + l5
    return total, (l1, l2, l3, l4, l5)


def kernel(pred_targets, gt_boxes, gt_labels):
    labf = gt_labels.astype(jnp.float32).reshape(gt_labels.shape[0], 8, 1)
    return _run(pred_targets, gt_boxes, labf)


# speedup vs baseline: 6.4187x; 1.7475x over previous
import functools

import jax
import jax.numpy as jnp
import numpy as np
from jax import lax
from jax.experimental import pallas as pl

_ANCHORS = np.array(
    [[1.3221, 1.73145], [3.19275, 4.00944], [5.05587, 8.09892],
     [9.47112, 4.84053], [11.2364, 10.0071]], dtype=np.float32)
_A = 5
_C = 20
_S = 13
_LOC = _S * _S * _A


def _anchor_rows_np():
    ys, xs = np.meshgrid(np.arange(_S, dtype=np.float32),
                         np.arange(_S, dtype=np.float32), indexing='ij')
    xy = np.stack([xs, ys], axis=-1) + 0.5
    xy = np.broadcast_to(xy[:, :, None, :], (_S, _S, _A, 2))
    wh = np.broadcast_to(_ANCHORS[None, None, :, :], (_S, _S, _A, 2))
    cat = np.concatenate([xy, wh], axis=-1).reshape(_LOC, 4)
    return np.ascontiguousarray(cat.T)


_ANC_ROWS = _anchor_rows_np()


_G = 8


def _tc_kernel(pred_ref, gt_ref, lab_ref, anc_ref, out_ref):
    b = pl.program_id(0)

    @pl.when(b == 0)
    def _init():
        out_ref[...] = jnp.zeros_like(out_ref)

    acc = jnp.zeros((8, 128), jnp.float32)
    for g in range(_G):
        acc = acc + _one_image(pred_ref[g], gt_ref[g], lab_ref[g],
                               anc_ref[...])
    out_ref[...] += acc


def _one_image(pt, gt, labf, anc):

    pxy = jax.nn.sigmoid(pt[0:2])
    pwh = jnp.exp(pt[2:4])
    pconf = jax.nn.sigmoid(pt[4:5])

    cpx = anc[0:1] + pxy[0:1]
    cpy = anc[1:2] + pxy[1:2]
    cpw = anc[2:3] * pwh[0:1]
    cph = anc[3:4] * pwh[1:2]
    px1 = cpx - cpw * 0.5
    py1 = cpy - cph * 0.5
    px2 = cpx + cpw * 0.5
    py2 = cpy + cph * 0.5
    area_p = (px2 - px1) * (py2 - py1)

    gx1 = gt[:, 0:1] * float(_S)
    gy1 = gt[:, 1:2] * float(_S)
    gx2 = gt[:, 2:3] * float(_S)
    gy2 = gt[:, 3:4] * float(_S)
    area_g = (gx2 - gx1) * (gy2 - gy1)

    tlx = jnp.maximum(px1, gx1)
    tly = jnp.maximum(py1, gy1)
    brx = jnp.minimum(px2, gx2)
    bry = jnp.minimum(py2, gy2)
    wx = jnp.maximum(brx - tlx, 0.0)
    wy = jnp.maximum(bry - tly, 0.0)
    inter = wx * wy
    iou = inter / (area_p + area_g - inter)
    gt_conf = jnp.max(iou, axis=0, keepdims=True)

    d = gt_conf - pconf
    dense_sum = jnp.sum(d * d)

    bx = (gx1 + gx2) * 0.5
    by = (gy1 + gy2) * 0.5
    bw = gx2 - gx1
    bh = gy2 - gy1
    cxf = jnp.floor(bx)
    cyf = jnp.floor(by)
    tx = bx - cxf
    ty = by - cyf

    best = jnp.full_like(bx, -1.0)
    jbest = jnp.zeros_like(bx)
    aw_j = jnp.zeros_like(bx)
    ah_j = jnp.zeros_like(bx)
    for a in range(_A):
        aw = float(_ANCHORS[a, 0])
        ah = float(_ANCHORS[a, 1])
        ax1 = cxf + 0.5 - aw * 0.5
        ay1 = cyf + 0.5 - ah * 0.5
        ax2 = cxf + 0.5 + aw * 0.5
        ay2 = cyf + 0.5 + ah * 0.5
        itlx = jnp.maximum(ax1, gx1)
        itly = jnp.maximum(ay1, gy1)
        ibrx = jnp.minimum(ax2, gx2)
        ibry = jnp.minimum(ay2, gy2)
        iwx = jnp.maximum(ibrx - itlx, 0.0)
        iwy = jnp.maximum(ibry - itly, 0.0)
        ai = iwx * iwy
        aiou = ai / (aw * ah + area_g - ai)
        take = aiou > best
        best = jnp.maximum(best, aiou)
        jbest = jnp.where(take, float(a), jbest)
        aw_j = jnp.where(take, aw, aw_j)
        ah_j = jnp.where(take, ah, ah_j)

    loc = (cyf * float(_S) + cxf) * float(_A) + jbest
    gw = bw / aw_j
    gh = bh / ah_j
    clsf = labf - 1.0

    loc_row = lax.dot_general(loc, jnp.eye(8, dtype=jnp.float32),
                              (((0,), (0,)), ((), ())))
    cls_row = lax.dot_general(clsf, jnp.eye(8, dtype=jnp.float32),
                              (((0,), (0,)), ((), ())))
    eq = (loc == loc_row)
    eq_pair = eq & (clsf == cls_row)
    n_iota = lax.broadcasted_iota(jnp.int32, (8, 8), 0).astype(jnp.float32)
    m_iota = lax.broadcasted_iota(jnp.int32, (8, 8), 1).astype(jnp.float32)
    later = m_iota > n_iota
    any_later_eq = jnp.sum(jnp.where(eq & later, 1.0, 0.0), axis=1,
                           keepdims=True)
    any_later_pair = jnp.sum(jnp.where(eq_pair & later, 1.0, 0.0), axis=1,
                             keepdims=True)
    winner = jnp.where(any_later_eq > 0.0, 0.0, 1.0)
    uniq = jnp.where(any_later_pair > 0.0, 0.0, 1.0)

    lane = lax.broadcasted_iota(jnp.int32, (8, _LOC), 1).astype(jnp.float32)
    onehot = jnp.where(lane == loc, 1.0, 0.0)
    g = lax.dot_general(onehot, pt, (((1,), (1,)), ((), ())))
    gtc = lax.dot_general(onehot, gt_conf, (((1,), (1,)), ((), ())))

    gxy = jax.nn.sigmoid(g[:, 0:2])
    gwh = jnp.exp(g[:, 2:4])
    gconf = jax.nn.sigmoid(g[:, 4:5])
    gcls = g[:, 5:25]

    txty = jnp.concatenate([tx, ty], axis=1)
    xy_s = jnp.sum(winner * (txty - gxy) ** 2)

    gtwh = jnp.concatenate([gw, gh], axis=1)
    wh_s = jnp.sum(winner * (jnp.sqrt(gtwh) - jnp.sqrt(gwh)) ** 2)

    conf_s = jnp.sum(winner * (gtc - gconf) ** 2)

    cmax = jnp.max(gcls, axis=1, keepdims=True)
    lse = jnp.log(jnp.sum(jnp.exp(gcls - cmax), axis=1, keepdims=True)) + cmax
    c_iota = lax.broadcasted_iota(jnp.int32, (8, _C), 1).astype(jnp.float32)
    sel = jnp.sum(jnp.where(c_iota == clsf, gcls, 0.0), axis=1, keepdims=True)
    cls_s = jnp.sum(uniq * (lse - sel))

    s_iota = lax.broadcasted_iota(jnp.int32, (8, 128), 0)
    l_iota = lax.broadcasted_iota(jnp.int32, (8, 128), 1)
    vals = (jnp.where(s_iota == 0, xy_s, 0.0)
            + jnp.where(s_iota == 1, wh_s, 0.0)
            + jnp.where(s_iota == 2, conf_s, 0.0)
            + jnp.where(s_iota == 3, dense_sum, 0.0)
            + jnp.where(s_iota == 4, cls_s, 0.0))
    return jnp.where(l_iota == 0, vals, 0.0)


@jax.jit
def _run(pred_targets, gt_boxes, labf):
    B = pred_targets.shape[0]
    predT = pred_targets.reshape(B, _S, _S, _A, 5 + _C)
    predT = predT.transpose(0, 4, 1, 2, 3).reshape(B, 5 + _C, _LOC)
    anc = jnp.asarray(_ANC_ROWS)

    out = pl.pallas_call(
        _tc_kernel,
        grid=(B // _G,),
        in_specs=[
            pl.BlockSpec((_G, 5 + _C, _LOC), lambda b: (b, 0, 0)),
            pl.BlockSpec((_G, 8, 4), lambda b: (b, 0, 0)),
            pl.BlockSpec((_G, 8, 1), lambda b: (b, 0, 0)),
            pl.BlockSpec((4, _LOC), lambda b: (0, 0)),
        ],
        out_specs=pl.BlockSpec((8, 128), lambda b: (0, 0)),
        out_shape=jax.ShapeDtypeStruct((8, 128), jnp.float32),
    )(predT, gt_boxes, labf, anc)

    o = out[:, 0]
    l1 = 5.0 * o[0]
    l2 = 5.0 * o[1]
    l3 = o[2]
    l4 = 0.5 * (o[3] - o[2])
    l5 = o[4]
    total = l1 + l2 + l3 + l4 + l5
    return total, (l1, l2, l3, l4, l5)


def kernel(pred_targets, gt_boxes, gt_labels):
    labf = gt_labels.astype(jnp.float32).reshape(gt_labels.shape[0], 8, 1)
    return _run(pred_targets, gt_boxes, labf)
